# baseline (speedup 1.0000x reference)
import functools

import jax
import jax.numpy as jnp
from jax import lax
from jax.experimental import pallas as pl
from jax.experimental.pallas import tpu as pltpu
from jax.experimental.pallas import tpu_sc as plsc

NC, NS = 2, 16
NW = NC * NS
N, E, D = 10000, 320000, 128
NB, CH = 125, 80

_mesh = plsc.VectorSubcoreMesh(core_axis_name="c", subcore_axis_name="s")


def _sc_degree(col3):
    @functools.partial(
        pl.kernel,
        out_type=jax.ShapeDtypeStruct((NC * N,), jnp.float32),
        mesh=_mesh,
        scratch_types=[
            pltpu.VMEM((NB, CH), jnp.int32),
            pltpu.VMEM((CH,), jnp.float32),
            pltpu.VMEM((N,), jnp.float32),
            pltpu.VMEM_SHARED((N,), jnp.float32),
            pltpu.SemaphoreType.DMA,
        ],
    )
    def deg_kernel(col_hbm, degp_hbm, cbuf2, ones, zbuf, degs, sem):
        cid = lax.axis_index("c")
        sid = lax.axis_index("s")
        wid = cid * NS + sid

        one_v = jnp.full((16,), 1.0, dtype=jnp.float32)
        zero_v = jnp.zeros((16,), dtype=jnp.float32)

        @pl.loop(0, CH // 16)
        def _(i):
            ones[pl.ds(i * 16, 16)] = one_v

        @pl.loop(0, N // 16)
        def _(i):
            zbuf[pl.ds(i * 16, 16)] = zero_v

        pltpu.sync_copy(zbuf, degs)
        plsc.subcore_barrier()

        pltpu.sync_copy(col_hbm.at[wid], cbuf2)

        @pl.loop(0, NB // 5)
        def _(p):
            for j in range(5):
                pltpu.make_async_copy(
                    ones, degs.at[cbuf2.at[5 * p + j]], sem).start(add=True)
            for j in range(5):
                pltpu.make_async_copy(
                    ones, degs.at[cbuf2.at[5 * p + j]], sem).wait()

        plsc.subcore_barrier()

        pltpu.sync_copy(degs, zbuf)
        pltpu.sync_copy(zbuf, degp_hbm.at[pl.ds(cid * N, N)])

    return deg_kernel(col3)


def _sc_aggregate(row1, col3, g):
    RPT = 624

    @functools.partial(
        pl.kernel,
        out_type=jax.ShapeDtypeStruct((NC * N, D), jnp.float32),
        mesh=_mesh,
        scratch_types=[
            pltpu.VMEM((NB * CH,), jnp.int32),
            pltpu.VMEM((NB, CH), jnp.int32),
            pltpu.VMEM((CH, D), jnp.float32),
            pltpu.VMEM((CH, D), jnp.float32),
            pltpu.VMEM_SHARED((N, D), jnp.float32),
            pltpu.SemaphoreType.DMA,
            pltpu.SemaphoreType.DMA,
            pltpu.SemaphoreType.DMA,
            pltpu.SemaphoreType.DMA,
        ],
    )
    def agg_kernel(row_hbm, col_hbm, g_hbm, raw_hbm, rbig, cbuf2,
                   gbuf0, gbuf1, aggs, sg0, sg1, ss0, ss1):
        cid = lax.axis_index("c")
        sid = lax.axis_index("s")
        wid = cid * NS + sid

        zero_v = jnp.zeros((16,), dtype=jnp.float32)

        @pl.loop(0, CH)
        def _(i):
            @pl.loop(0, D // 16)
            def _(j):
                gbuf0[i, pl.ds(j * 16, 16)] = zero_v

        row0 = sid * RPT
        for k in range(7):
            pltpu.sync_copy(gbuf0, aggs.at[pl.ds(row0 + k * 80, 80)])
        pltpu.sync_copy(gbuf0.at[pl.ds(0, 64)],
                        aggs.at[pl.ds(row0 + 560, 64)])
        pltpu.sync_copy(gbuf0.at[pl.ds(0, 16)], aggs.at[pl.ds(NS * RPT, 16)])

        plsc.subcore_barrier()

        pltpu.sync_copy(row_hbm.at[pl.ds(wid * NB * CH, NB * CH)], rbig)
        pltpu.sync_copy(col_hbm.at[wid], cbuf2)

        def gather(b, buf, sem):
            idx = rbig.at[pl.ds(b * CH, CH)]
            return pltpu.make_async_copy(g_hbm.at[idx], buf, sem)

        def scat(b, buf, sem):
            return pltpu.make_async_copy(buf, aggs.at[cbuf2.at[b]], sem)

        gather(0, gbuf0, sg0).start()
        gather(1, gbuf1, sg1).start()

        @pl.loop(0, (NB - 3) // 2)
        def _(p):
            b0 = 2 * p
            gather(b0, gbuf0, sg0).wait()
            scat(b0, gbuf0, ss0).start(add=True)
            gather(b0 + 1, gbuf1, sg1).wait()
            scat(b0, gbuf0, ss0).wait()
            scat(b0 + 1, gbuf1, ss1).start(add=True)
            gather(b0 + 2, gbuf0, sg0).start()
            scat(b0 + 1, gbuf1, ss1).wait()
            gather(b0 + 3, gbuf1, sg1).start()

        gather(NB - 3, gbuf0, sg0).wait()
        scat(NB - 3, gbuf0, ss0).start(add=True)
        gather(NB - 2, gbuf1, sg1).wait()
        scat(NB - 3, gbuf0, ss0).wait()
        scat(NB - 2, gbuf1, ss1).start(add=True)
        gather(NB - 1, gbuf0, sg0).start()
        scat(NB - 2, gbuf1, ss1).wait()
        gather(NB - 1, gbuf0, sg0).wait()
        scat(NB - 1, gbuf0, ss0).start(add=True)
        scat(NB - 1, gbuf0, ss0).wait()

        plsc.subcore_barrier()

        out0 = cid * N
        for k in range(7):
            r = row0 + k * 80
            pltpu.sync_copy(aggs.at[pl.ds(r, 80)], gbuf0)
            pltpu.sync_copy(gbuf0, raw_hbm.at[pl.ds(out0 + r, 80)])
        pltpu.sync_copy(aggs.at[pl.ds(row0 + 560, 64)], gbuf0.at[pl.ds(0, 64)])
        pltpu.sync_copy(gbuf0.at[pl.ds(0, 64)],
                        raw_hbm.at[pl.ds(out0 + row0 + 560, 64)])
        pltpu.sync_copy(aggs.at[pl.ds(NS * RPT, 16)], gbuf0.at[pl.ds(0, 16)])
        pltpu.sync_copy(gbuf0.at[pl.ds(0, 16)],
                        raw_hbm.at[pl.ds(out0 + NS * RPT, 16)])

    return agg_kernel(row1, col3, g)


_GRID = 5
_BM = N // _GRID


def _dis_of(d_ref):
    deg = d_ref[0, :, 0:1] + d_ref[1, :, 0:1]
    return jnp.where(deg > 0, lax.rsqrt(deg), 0.0)


def _mm0g(x, W0, b0, degp):
    def body(x_ref, w_ref, b_ref, d_ref, h_ref, g_ref):
        h = jnp.dot(x_ref[...], w_ref[...],
                    preferred_element_type=jnp.float32) + b_ref[...]
        h_ref[...] = h
        g_ref[...] = h * _dis_of(d_ref)

    return pl.pallas_call(
        body,
        grid=(_GRID,),
        in_specs=[
            pl.BlockSpec((_BM, D), lambda i: (i, 0)),
            pl.BlockSpec((D, D), lambda i: (0, 0)),
            pl.BlockSpec((1, D), lambda i: (0, 0)),
            pl.BlockSpec((NC, _BM, 16), lambda i: (0, i, 0)),
        ],
        out_specs=[pl.BlockSpec((_BM, D), lambda i: (i, 0)),
                   pl.BlockSpec((_BM, D), lambda i: (i, 0))],
        out_shape=[jax.ShapeDtypeStruct((N, D), jnp.float32),
                   jax.ShapeDtypeStruct((N, D), jnp.float32)],
    )(x, W0, b0.reshape(1, D), degp)


def _finale(rawp, degp, h, W1, b1):
    def body(r_ref, d_ref, h_ref, w_ref, b_ref, o_ref):
        agg = r_ref[0] + r_ref[1]
        a = 0.5 * (_dis_of(d_ref) * agg) + 0.5 * h_ref[...]
        nrm = jnp.sqrt(jnp.sum(a * a, axis=1, keepdims=True))
        a = a / jnp.maximum(nrm, 1e-12)
        a = jnp.maximum(a, 0.0)
        o_ref[...] = jnp.dot(a, w_ref[...],
                             preferred_element_type=jnp.float32) + b_ref[...]

    return pl.pallas_call(
        body,
        grid=(_GRID,),
        in_specs=[
            pl.BlockSpec((NC, _BM, D), lambda i: (0, i, 0)),
            pl.BlockSpec((NC, _BM, 16), lambda i: (0, i, 0)),
            pl.BlockSpec((_BM, D), lambda i: (i, 0)),
            pl.BlockSpec((D, D), lambda i: (0, 0)),
            pl.BlockSpec((1, D), lambda i: (0, 0)),
        ],
        out_specs=pl.BlockSpec((_BM, D), lambda i: (i, 0)),
        out_shape=jax.ShapeDtypeStruct((N, D), jnp.float32),
    )(rawp, degp, h, W1, b1.reshape(1, D))


def kernel(x, edge_index, W0, b0, W1, b1):
    row1 = edge_index[0]
    col3 = edge_index[1].reshape(NW, NB, CH)
    degf = _sc_degree(col3)
    degp = jnp.broadcast_to(degf.reshape(NC, N, 1), (NC, N, 16))
    h, g = _mm0g(x, W0, b0, degp)
    rawp = _sc_aggregate(row1, col3, g).reshape(NC, N, D)
    return _finale(rawp, degp, h, W1, b1)


# speedup vs baseline: 1.0255x; 1.0255x over previous
import functools

import jax
import jax.numpy as jnp
from jax import lax
from jax.experimental import pallas as pl
from jax.experimental.pallas import tpu as pltpu
from jax.experimental.pallas import tpu_sc as plsc

NC, NS = 2, 16
NW = NC * NS
N, E, D = 10000, 320000, 128
NB, CH = 125, 80

_mesh = plsc.VectorSubcoreMesh(core_axis_name="c", subcore_axis_name="s")


def _sc_degree(col3):
    @functools.partial(
        pl.kernel,
        out_type=jax.ShapeDtypeStruct((NC * N,), jnp.float32),
        mesh=_mesh,
        scratch_types=[
            pltpu.VMEM((NB, CH), jnp.int32),
            pltpu.VMEM((CH,), jnp.float32),
            pltpu.VMEM((N,), jnp.float32),
            pltpu.VMEM_SHARED((N,), jnp.float32),
            pltpu.SemaphoreType.DMA,
        ],
    )
    def deg_kernel(col_hbm, degp_hbm, cbuf2, ones, zbuf, degs, sem):
        cid = lax.axis_index("c")
        sid = lax.axis_index("s")
        wid = cid * NS + sid

        one_v = jnp.full((16,), 1.0, dtype=jnp.float32)
        zero_v = jnp.zeros((16,), dtype=jnp.float32)

        @pl.loop(0, CH // 16)
        def _(i):
            ones[pl.ds(i * 16, 16)] = one_v

        @pl.loop(0, N // 16)
        def _(i):
            zbuf[pl.ds(i * 16, 16)] = zero_v

        pltpu.sync_copy(zbuf, degs)
        plsc.subcore_barrier()

        pltpu.sync_copy(col_hbm.at[wid], cbuf2)

        @pl.loop(0, NB // 5)
        def _(p):
            for j in range(5):
                pltpu.make_async_copy(
                    ones, degs.at[cbuf2.at[5 * p + j]], sem).start(add=True)
            for j in range(5):
                pltpu.make_async_copy(
                    ones, degs.at[cbuf2.at[5 * p + j]], sem).wait()

        plsc.subcore_barrier()

        pltpu.sync_copy(degs, zbuf)
        pltpu.sync_copy(zbuf, degp_hbm.at[pl.ds(cid * N, N)])

    return deg_kernel(col3)


def _sc_aggregate(row1, col3, g):
    RPT = 624

    @functools.partial(
        pl.kernel,
        out_type=jax.ShapeDtypeStruct((NC * N, D), jnp.float32),
        mesh=_mesh,
        scratch_types=[
            pltpu.VMEM((NB * CH,), jnp.int32),
            pltpu.VMEM((NB, CH), jnp.int32),
            pltpu.VMEM((CH, D), jnp.float32),
            pltpu.VMEM((CH, D), jnp.float32),
            pltpu.VMEM_SHARED((N, D), jnp.float32),
            pltpu.SemaphoreType.DMA,
            pltpu.SemaphoreType.DMA,
            pltpu.SemaphoreType.DMA,
            pltpu.SemaphoreType.DMA,
        ],
    )
    def agg_kernel(row_hbm, col_hbm, g_hbm, raw_hbm, rbig, cbuf2,
                   gbuf0, gbuf1, aggs, sg0, sg1, ss0, ss1):
        cid = lax.axis_index("c")
        sid = lax.axis_index("s")
        wid = cid * NS + sid

        zero_v = jnp.zeros((16,), dtype=jnp.float32)

        ld_r = pltpu.make_async_copy(
            row_hbm.at[pl.ds(wid * NB * CH, NB * CH)], rbig, sg0)
        ld_c = pltpu.make_async_copy(col_hbm.at[wid], cbuf2, sg1)
        ld_r.start()
        ld_c.start()

        @pl.loop(0, CH)
        def _(i):
            @pl.loop(0, D // 16)
            def _(j):
                gbuf0[i, pl.ds(j * 16, 16)] = zero_v

        row0 = sid * RPT
        zcopies = [(gbuf0, aggs.at[pl.ds(row0 + k * 80, 80)]) for k in range(7)]
        zcopies.append((gbuf0.at[pl.ds(0, 64)], aggs.at[pl.ds(row0 + 560, 64)]))
        zcopies.append((gbuf0.at[pl.ds(0, 16)], aggs.at[pl.ds(NS * RPT, 16)]))
        for grp in (zcopies[:5], zcopies[5:]):
            for s, d in grp:
                pltpu.make_async_copy(s, d, ss0).start()
            for s, d in grp:
                pltpu.make_async_copy(s, d, ss0).wait()

        ld_r.wait()
        ld_c.wait()

        plsc.subcore_barrier()

        def gather(b, buf, sem):
            idx = rbig.at[pl.ds(b * CH, CH)]
            return pltpu.make_async_copy(g_hbm.at[idx], buf, sem)

        def scat(b, buf, sem):
            return pltpu.make_async_copy(buf, aggs.at[cbuf2.at[b]], sem)

        gather(0, gbuf0, sg0).start()
        gather(1, gbuf1, sg1).start()

        @pl.loop(0, (NB - 3) // 2)
        def _(p):
            b0 = 2 * p
            gather(b0, gbuf0, sg0).wait()
            scat(b0, gbuf0, ss0).start(add=True)
            gather(b0 + 1, gbuf1, sg1).wait()
            scat(b0, gbuf0, ss0).wait()
            scat(b0 + 1, gbuf1, ss1).start(add=True)
            gather(b0 + 2, gbuf0, sg0).start()
            scat(b0 + 1, gbuf1, ss1).wait()
            gather(b0 + 3, gbuf1, sg1).start()

        gather(NB - 3, gbuf0, sg0).wait()
        scat(NB - 3, gbuf0, ss0).start(add=True)
        gather(NB - 2, gbuf1, sg1).wait()
        scat(NB - 3, gbuf0, ss0).wait()
        scat(NB - 2, gbuf1, ss1).start(add=True)
        gather(NB - 1, gbuf0, sg0).start()
        scat(NB - 2, gbuf1, ss1).wait()
        gather(NB - 1, gbuf0, sg0).wait()
        scat(NB - 1, gbuf0, ss0).start(add=True)
        scat(NB - 1, gbuf0, ss0).wait()

        plsc.subcore_barrier()

        out0 = cid * N
        chunks = [(row0 + k * 80, 80) for k in range(7)]
        chunks.append((row0 + 560, 64))
        chunks.append((NS * RPT, 16))
        bufs = (gbuf0, gbuf1)
        sems = (ss0, ss1)
        pend = [None, None]
        for k, (r, n) in enumerate(chunks):
            buf, sem = bufs[k % 2], sems[k % 2]
            if pend[k % 2] is not None:
                pltpu.make_async_copy(*pend[k % 2], sem).wait()
            pltpu.sync_copy(aggs.at[pl.ds(r, n)], buf.at[pl.ds(0, n)])
            pltpu.make_async_copy(
                buf.at[pl.ds(0, n)], raw_hbm.at[pl.ds(out0 + r, n)], sem
            ).start()
            pend[k % 2] = (buf.at[pl.ds(0, n)],
                           raw_hbm.at[pl.ds(out0 + r, n)])
        for k in range(2):
            if pend[k] is not None:
                pltpu.make_async_copy(*pend[k], sems[k]).wait()

    return agg_kernel(row1, col3, g)


_GRID = 5
_BM = N // _GRID


def _dis_of(d_ref):
    deg = d_ref[0, :, 0:1] + d_ref[1, :, 0:1]
    return jnp.where(deg > 0, lax.rsqrt(deg), 0.0)


def _mm0g(x, W0, b0, degp):
    def body(x_ref, w_ref, b_ref, d_ref, h_ref, g_ref):
        h = jnp.dot(x_ref[...], w_ref[...],
                    preferred_element_type=jnp.float32) + b_ref[...]
        h_ref[...] = h
        g_ref[...] = h * _dis_of(d_ref)

    return pl.pallas_call(
        body,
        grid=(_GRID,),
        in_specs=[
            pl.BlockSpec((_BM, D), lambda i: (i, 0)),
            pl.BlockSpec((D, D), lambda i: (0, 0)),
            pl.BlockSpec((1, D), lambda i: (0, 0)),
            pl.BlockSpec((NC, _BM, 16), lambda i: (0, i, 0)),
        ],
        out_specs=[pl.BlockSpec((_BM, D), lambda i: (i, 0)),
                   pl.BlockSpec((_BM, D), lambda i: (i, 0))],
        out_shape=[jax.ShapeDtypeStruct((N, D), jnp.float32),
                   jax.ShapeDtypeStruct((N, D), jnp.float32)],
    )(x, W0, b0.reshape(1, D), degp)


def _finale(rawp, degp, h, W1, b1):
    def body(r_ref, d_ref, h_ref, w_ref, b_ref, o_ref):
        agg = r_ref[0] + r_ref[1]
        a = 0.5 * (_dis_of(d_ref) * agg) + 0.5 * h_ref[...]
        nrm = jnp.sqrt(jnp.sum(a * a, axis=1, keepdims=True))
        a = a / jnp.maximum(nrm, 1e-12)
        a = jnp.maximum(a, 0.0)
        o_ref[...] = jnp.dot(a, w_ref[...],
                             preferred_element_type=jnp.float32) + b_ref[...]

    return pl.pallas_call(
        body,
        grid=(_GRID,),
        in_specs=[
            pl.BlockSpec((NC, _BM, D), lambda i: (0, i, 0)),
            pl.BlockSpec((NC, _BM, 16), lambda i: (0, i, 0)),
            pl.BlockSpec((_BM, D), lambda i: (i, 0)),
            pl.BlockSpec((D, D), lambda i: (0, 0)),
            pl.BlockSpec((1, D), lambda i: (0, 0)),
        ],
        out_specs=pl.BlockSpec((_BM, D), lambda i: (i, 0)),
        out_shape=jax.ShapeDtypeStruct((N, D), jnp.float32),
    )(rawp, degp, h, W1, b1.reshape(1, D))


def kernel(x, edge_index, W0, b0, W1, b1):
    row1 = edge_index[0]
    col3 = edge_index[1].reshape(NW, NB, CH)
    degf = _sc_degree(col3)
    degp = jnp.broadcast_to(degf.reshape(NC, N, 1), (NC, N, 16))
    h, g = _mm0g(x, W0, b0, degp)
    rawp = _sc_aggregate(row1, col3, g).reshape(NC, N, D)
    return _finale(rawp, degp, h, W1, b1)


# speedup vs baseline: 1.0296x; 1.0040x over previous
import functools

import jax
import jax.numpy as jnp
from jax import lax
from jax.experimental import pallas as pl
from jax.experimental.pallas import tpu as pltpu
from jax.experimental.pallas import tpu_sc as plsc

NC, NS = 2, 16
NW = NC * NS
N, E, D = 10000, 320000, 128
NB, CH = 125, 80

_mesh = plsc.VectorSubcoreMesh(core_axis_name="c", subcore_axis_name="s")


def _sc_degree(col3):
    @functools.partial(
        pl.kernel,
        out_type=jax.ShapeDtypeStruct((NC * N,), jnp.float32),
        mesh=_mesh,
        scratch_types=[
            pltpu.VMEM((NB, CH), jnp.int32),
            pltpu.VMEM((CH,), jnp.float32),
            pltpu.VMEM((N,), jnp.float32),
            pltpu.VMEM_SHARED((N,), jnp.float32),
            pltpu.SemaphoreType.DMA,
        ],
    )
    def deg_kernel(col_hbm, degp_hbm, cbuf2, ones, zbuf, degs, sem):
        cid = lax.axis_index("c")
        sid = lax.axis_index("s")
        wid = cid * NS + sid

        one_v = jnp.full((16,), 1.0, dtype=jnp.float32)
        zero_v = jnp.zeros((16,), dtype=jnp.float32)

        @pl.loop(0, CH // 16)
        def _(i):
            ones[pl.ds(i * 16, 16)] = one_v

        @pl.loop(0, N // 16)
        def _(i):
            zbuf[pl.ds(i * 16, 16)] = zero_v

        pltpu.sync_copy(zbuf, degs)
        plsc.subcore_barrier()

        pltpu.sync_copy(col_hbm.at[wid], cbuf2)

        @pl.loop(0, 15)
        def _(p):
            for j in range(8):
                pltpu.make_async_copy(
                    ones, degs.at[cbuf2.at[8 * p + j]], sem).start(add=True)
            for j in range(8):
                pltpu.make_async_copy(
                    ones, degs.at[cbuf2.at[8 * p + j]], sem).wait()

        for j in range(120, NB):
            pltpu.make_async_copy(
                ones, degs.at[cbuf2.at[j]], sem).start(add=True)
        for j in range(120, NB):
            pltpu.make_async_copy(
                ones, degs.at[cbuf2.at[j]], sem).wait()

        plsc.subcore_barrier()

        pltpu.sync_copy(degs, zbuf)
        pltpu.sync_copy(zbuf, degp_hbm.at[pl.ds(cid * N, N)])

    return deg_kernel(col3)


def _sc_aggregate(row1, col3, g):
    RPT = 624

    @functools.partial(
        pl.kernel,
        out_type=jax.ShapeDtypeStruct((NC * N, D), jnp.float32),
        mesh=_mesh,
        scratch_types=[
            pltpu.VMEM((NB * CH,), jnp.int32),
            pltpu.VMEM((NB, CH), jnp.int32),
            pltpu.VMEM((CH, D), jnp.float32),
            pltpu.VMEM((CH, D), jnp.float32),
            pltpu.VMEM_SHARED((N, D), jnp.float32),
            pltpu.SemaphoreType.DMA,
            pltpu.SemaphoreType.DMA,
            pltpu.SemaphoreType.DMA,
            pltpu.SemaphoreType.DMA,
        ],
    )
    def agg_kernel(row_hbm, col_hbm, g_hbm, raw_hbm, rbig, cbuf2,
                   gbuf0, gbuf1, aggs, sg0, sg1, ss0, ss1):
        cid = lax.axis_index("c")
        sid = lax.axis_index("s")
        wid = cid * NS + sid

        zero_v = jnp.zeros((16,), dtype=jnp.float32)

        ld_r = pltpu.make_async_copy(
            row_hbm.at[pl.ds(wid * NB * CH, NB * CH)], rbig, sg0)
        ld_c = pltpu.make_async_copy(col_hbm.at[wid], cbuf2, sg1)
        ld_r.start()
        ld_c.start()

        @pl.loop(0, CH)
        def _(i):
            @pl.loop(0, D // 16)
            def _(j):
                gbuf0[i, pl.ds(j * 16, 16)] = zero_v

        row0 = sid * RPT
        zcopies = [(gbuf0, aggs.at[pl.ds(row0 + k * 80, 80)]) for k in range(7)]
        zcopies.append((gbuf0.at[pl.ds(0, 64)], aggs.at[pl.ds(row0 + 560, 64)]))
        zcopies.append((gbuf0.at[pl.ds(0, 16)], aggs.at[pl.ds(NS * RPT, 16)]))
        for grp in (zcopies[:5], zcopies[5:]):
            for s, d in grp:
                pltpu.make_async_copy(s, d, ss0).start()
            for s, d in grp:
                pltpu.make_async_copy(s, d, ss0).wait()

        ld_r.wait()
        ld_c.wait()

        plsc.subcore_barrier()

        def gather(b, buf, sem):
            idx = rbig.at[pl.ds(b * CH, CH)]
            return pltpu.make_async_copy(g_hbm.at[idx], buf, sem)

        def scat(b, buf, sem):
            return pltpu.make_async_copy(buf, aggs.at[cbuf2.at[b]], sem)

        gather(0, gbuf0, sg0).start()
        gather(1, gbuf1, sg1).start()

        @pl.loop(0, (NB - 3) // 2)
        def _(p):
            b0 = 2 * p
            gather(b0, gbuf0, sg0).wait()
            scat(b0, gbuf0, ss0).start(add=True)
            gather(b0 + 1, gbuf1, sg1).wait()
            scat(b0, gbuf0, ss0).wait()
            scat(b0 + 1, gbuf1, ss1).start(add=True)
            gather(b0 + 2, gbuf0, sg0).start()
            scat(b0 + 1, gbuf1, ss1).wait()
            gather(b0 + 3, gbuf1, sg1).start()

        gather(NB - 3, gbuf0, sg0).wait()
        scat(NB - 3, gbuf0, ss0).start(add=True)
        gather(NB - 2, gbuf1, sg1).wait()
        scat(NB - 3, gbuf0, ss0).wait()
        scat(NB - 2, gbuf1, ss1).start(add=True)
        gather(NB - 1, gbuf0, sg0).start()
        scat(NB - 2, gbuf1, ss1).wait()
        gather(NB - 1, gbuf0, sg0).wait()
        scat(NB - 1, gbuf0, ss0).start(add=True)
        scat(NB - 1, gbuf0, ss0).wait()

        plsc.subcore_barrier()

        out0 = cid * N
        chunks = [(row0 + k * 80, 80) for k in range(7)]
        chunks.append((row0 + 560, 64))
        chunks.append((NS * RPT, 16))
        bufs = (gbuf0, gbuf1)
        sems = (ss0, ss1)
        pend = [None, None]
        for k, (r, n) in enumerate(chunks):
            buf, sem = bufs[k % 2], sems[k % 2]
            if pend[k % 2] is not None:
                pltpu.make_async_copy(*pend[k % 2], sem).wait()
            pltpu.sync_copy(aggs.at[pl.ds(r, n)], buf.at[pl.ds(0, n)])
            pltpu.make_async_copy(
                buf.at[pl.ds(0, n)], raw_hbm.at[pl.ds(out0 + r, n)], sem
            ).start()
            pend[k % 2] = (buf.at[pl.ds(0, n)],
                           raw_hbm.at[pl.ds(out0 + r, n)])
        for k in range(2):
            if pend[k] is not None:
                pltpu.make_async_copy(*pend[k], sems[k]).wait()

    return agg_kernel(row1, col3, g)


_GRID = 5
_BM = N // _GRID


def _dis_of(d_ref):
    deg = d_ref[0, :, 0:1] + d_ref[1, :, 0:1]
    return jnp.where(deg > 0, lax.rsqrt(deg), 0.0)


def _mm0g(x, W0, b0, degp):
    def body(x_ref, w_ref, b_ref, d_ref, h_ref, g_ref):
        h = jnp.dot(x_ref[...], w_ref[...],
                    preferred_element_type=jnp.float32) + b_ref[...]
        h_ref[...] = h
        g_ref[...] = h * _dis_of(d_ref)

    return pl.pallas_call(
        body,
        grid=(_GRID,),
        in_specs=[
            pl.BlockSpec((_BM, D), lambda i: (i, 0)),
            pl.BlockSpec((D, D), lambda i: (0, 0)),
            pl.BlockSpec((1, D), lambda i: (0, 0)),
            pl.BlockSpec((NC, _BM, 16), lambda i: (0, i, 0)),
        ],
        out_specs=[pl.BlockSpec((_BM, D), lambda i: (i, 0)),
                   pl.BlockSpec((_BM, D), lambda i: (i, 0))],
        out_shape=[jax.ShapeDtypeStruct((N, D), jnp.float32),
                   jax.ShapeDtypeStruct((N, D), jnp.float32)],
    )(x, W0, b0.reshape(1, D), degp)


def _finale(rawp, degp, h, W1, b1):
    def body(r_ref, d_ref, h_ref, w_ref, b_ref, o_ref):
        agg = r_ref[0] + r_ref[1]
        a = 0.5 * (_dis_of(d_ref) * agg) + 0.5 * h_ref[...]
        nrm = jnp.sqrt(jnp.sum(a * a, axis=1, keepdims=True))
        a = a / jnp.maximum(nrm, 1e-12)
        a = jnp.maximum(a, 0.0)
        o_ref[...] = jnp.dot(a, w_ref[...],
                             preferred_element_type=jnp.float32) + b_ref[...]

    return pl.pallas_call(
        body,
        grid=(_GRID,),
        in_specs=[
            pl.BlockSpec((NC, _BM, D), lambda i: (0, i, 0)),
            pl.BlockSpec((NC, _BM, 16), lambda i: (0, i, 0)),
            pl.BlockSpec((_BM, D), lambda i: (i, 0)),
            pl.BlockSpec((D, D), lambda i: (0, 0)),
            pl.BlockSpec((1, D), lambda i: (0, 0)),
        ],
        out_specs=pl.BlockSpec((_BM, D), lambda i: (i, 0)),
        out_shape=jax.ShapeDtypeStruct((N, D), jnp.float32),
    )(rawp, degp, h, W1, b1.reshape(1, D))


def kernel(x, edge_index, W0, b0, W1, b1):
    row1 = edge_index[0]
    col3 = edge_index[1].reshape(NW, NB, CH)
    degf = _sc_degree(col3)
    degp = jnp.broadcast_to(degf.reshape(NC, N, 1), (NC, N, 16))
    h, g = _mm0g(x, W0, b0, degp)
    rawp = _sc_aggregate(row1, col3, g).reshape(NC, N, D)
    return _finale(rawp, degp, h, W1, b1)
